# baseline (speedup 1.0000x reference)
import jax
import jax.numpy as jnp
from jax.experimental import pallas as pl

PAD = 0
B, S = 4096, 50
D = 64
H2 = 256
N = B * S
EPS = 1e-12
ROWS = 2048


def _ln_up_body(e_ref, g_ref, b_ref, w_ref, ub_ref, o_ref):
    e = e_ref[...]
    mu = jnp.mean(e, axis=1, keepdims=True)
    c = e - mu
    var = jnp.mean(c * c, axis=1, keepdims=True)
    normed = c * jax.lax.rsqrt(var + EPS) * g_ref[...] + b_ref[...]
    o_ref[...] = (
        jnp.dot(normed, w_ref[...], preferred_element_type=jnp.float32)
        + ub_ref[...]
    )


def _ln_up(e2, ln_gamma, ln_beta, up_w, up_b):
    grid = (N // ROWS,)
    return pl.pallas_call(
        _ln_up_body,
        grid=grid,
        in_specs=[
            pl.BlockSpec((ROWS, D), lambda i: (i, 0)),
            pl.BlockSpec((1, D), lambda i: (0, 0)),
            pl.BlockSpec((1, D), lambda i: (0, 0)),
            pl.BlockSpec((D, H2), lambda i: (0, 0)),
            pl.BlockSpec((1, H2), lambda i: (0, 0)),
        ],
        out_specs=pl.BlockSpec((ROWS, H2), lambda i: (i, 0)),
        out_shape=jax.ShapeDtypeStruct((N, H2), jnp.float32),
    )(e2, ln_gamma.reshape(1, D), ln_beta.reshape(1, D), up_w,
      up_b.reshape(1, H2))


def kernel(src, length, aa_table, pos_table, pos2_table, ln_gamma, ln_beta,
           up_w, up_b):
    src = src.astype(jnp.int32)
    length = length.astype(jnp.int32)
    mask = (src != PAD).astype(jnp.int32)
    pid = jnp.cumsum(mask, axis=1) * mask
    pid2 = jnp.clip((length[:, None] - pid + 2) * mask, 0, pos2_table.shape[0] - 1)
    e = (jnp.take(aa_table, src, axis=0)
         + jnp.take(pos_table, pid, axis=0)
         + jnp.take(pos2_table, pid2, axis=0))
    out = _ln_up(e.reshape(N, D), ln_gamma, ln_beta, up_w, up_b)
    return out.reshape(B, S, H2)


# speedup vs baseline: 2.9638x; 2.9638x over previous
import functools

import jax
import jax.numpy as jnp
from jax import lax
from jax.experimental import pallas as pl
from jax.experimental.pallas import tpu as pltpu
from jax.experimental.pallas import tpu_sc as plsc

PAD = 0
B, S = 4096, 50
D = 64
H2 = 256
N = B * S
EPS = 1e-12

IDX_ROWS = 512


def _idx_body(src_ref, len_ref, pid_ref, pid2_ref):
    src = src_ref[...]
    m = src != PAD
    mf = m.astype(jnp.float32)
    r = lax.broadcasted_iota(jnp.int32, (S, S), 0)
    c = lax.broadcasted_iota(jnp.int32, (S, S), 1)
    tri = (r <= c).astype(jnp.float32)
    pid = jnp.dot(mf, tri, preferred_element_type=jnp.float32).astype(jnp.int32)
    mi = m.astype(jnp.int32)
    pid = pid * mi
    p2 = jnp.maximum(len_ref[...] + 2 - pid, 0) * mi
    pid_ref[...] = pid
    pid2_ref[...] = p2


def _make_ids(src, length):
    grid = (B // IDX_ROWS,)
    return pl.pallas_call(
        _idx_body,
        grid=grid,
        in_specs=[
            pl.BlockSpec((IDX_ROWS, S), lambda i: (i, 0)),
            pl.BlockSpec((IDX_ROWS, 1), lambda i: (i, 0)),
        ],
        out_specs=(
            pl.BlockSpec((IDX_ROWS, S), lambda i: (i, 0)),
            pl.BlockSpec((IDX_ROWS, S), lambda i: (i, 0)),
        ),
        out_shape=(
            jax.ShapeDtypeStruct((B, S), jnp.int32),
            jax.ShapeDtypeStruct((B, S), jnp.int32),
        ),
    )(src, length.reshape(B, 1))


NW = 32
TPW = N // NW
CHUNK = 128
NCHUNK = TPW // CHUNK


def _sc_gather_body(aa_idx, pos_idx, pos2_idx, aa_t, pos_t, pos2_t, out,
                    idx_a, idx_b, idx_c, buf_a, buf_b, buf_c, buf_e,
                    sem_a, sem_b, sem_c):
    wid = lax.axis_index("s") * 2 + lax.axis_index("c")
    base = wid * TPW

    def chunk_body(it, carry):
        tb = base + it * CHUNK
        pltpu.sync_copy(aa_idx.at[pl.ds(tb, CHUNK)], idx_a)
        pltpu.sync_copy(pos_idx.at[pl.ds(tb, CHUNK)], idx_b)
        pltpu.sync_copy(pos2_idx.at[pl.ds(tb, CHUNK)], idx_c)
        cp_a = pltpu.async_copy(aa_t.at[idx_a], buf_a, sem_a)
        cp_b = pltpu.async_copy(pos_t.at[idx_b], buf_b, sem_b)
        cp_c = pltpu.async_copy(pos2_t.at[idx_c], buf_c, sem_c)
        cp_a.wait()
        cp_b.wait()
        cp_c.wait()

        def add_body(j, carry2):
            for rr in range(2):
                for cc in range(D // 16):
                    sl = (2 * j + rr, pl.ds(cc * 16, 16))
                    buf_e[sl] = buf_a[sl] + buf_b[sl] + buf_c[sl]
            return carry2

        lax.fori_loop(0, CHUNK // 2, add_body, 0, unroll=False)
        pltpu.sync_copy(buf_e, out.at[pl.ds(tb, CHUNK)])
        return carry

    lax.fori_loop(0, NCHUNK, chunk_body, 0, unroll=False)


_sc_gather = functools.partial(
    pl.kernel,
    out_type=jax.ShapeDtypeStruct((N, D), jnp.float32),
    mesh=plsc.VectorSubcoreMesh(core_axis_name="c", subcore_axis_name="s"),
    scratch_types=[
        pltpu.VMEM((CHUNK,), jnp.int32),
        pltpu.VMEM((CHUNK,), jnp.int32),
        pltpu.VMEM((CHUNK,), jnp.int32),
        pltpu.VMEM((CHUNK, 2 * D), jnp.float32),
        pltpu.VMEM((CHUNK, 2 * D), jnp.float32),
        pltpu.VMEM((CHUNK, 2 * D), jnp.float32),
        pltpu.VMEM((CHUNK, D), jnp.float32),
        pltpu.SemaphoreType.DMA,
        pltpu.SemaphoreType.DMA,
        pltpu.SemaphoreType.DMA,
    ],
)(_sc_gather_body)


ROWS = 2048


def _ln_up_body(e_ref, g_ref, b_ref, w_ref, ub_ref, o_ref):
    e = e_ref[...]
    mu = jnp.mean(e, axis=1, keepdims=True)
    cent = e - mu
    var = jnp.mean(cent * cent, axis=1, keepdims=True)
    normed = cent * lax.rsqrt(var + EPS) * g_ref[...] + b_ref[...]
    o_ref[...] = (
        jnp.dot(normed, w_ref[...], preferred_element_type=jnp.float32)
        + ub_ref[...]
    )


def _ln_up(e2, ln_gamma, ln_beta, up_w, up_b):
    grid = (N // ROWS,)
    return pl.pallas_call(
        _ln_up_body,
        grid=grid,
        in_specs=[
            pl.BlockSpec((ROWS, D), lambda i: (i, 0)),
            pl.BlockSpec((1, D), lambda i: (0, 0)),
            pl.BlockSpec((1, D), lambda i: (0, 0)),
            pl.BlockSpec((D, H2), lambda i: (0, 0)),
            pl.BlockSpec((1, H2), lambda i: (0, 0)),
        ],
        out_specs=pl.BlockSpec((ROWS, H2), lambda i: (i, 0)),
        out_shape=jax.ShapeDtypeStruct((N, H2), jnp.float32),
    )(e2, ln_gamma.reshape(1, D), ln_beta.reshape(1, D), up_w,
      up_b.reshape(1, H2))


def kernel(src, length, aa_table, pos_table, pos2_table, ln_gamma, ln_beta,
           up_w, up_b):
    src = src.astype(jnp.int32)
    length = length.astype(jnp.int32)
    pid, pid2 = _make_ids(src, length)
    pad = ((0, 0), (0, D))
    e = _sc_gather(src.reshape(N), pid.reshape(N), pid2.reshape(N),
                   jnp.pad(aa_table, pad), jnp.pad(pos_table, pad),
                   jnp.pad(pos2_table, pad))
    out = _ln_up(e, ln_gamma, ln_beta, up_w, up_b)
    return out.reshape(B, S, H2)


# speedup vs baseline: 3.2353x; 1.0916x over previous
import functools

import jax
import jax.numpy as jnp
from jax import lax
from jax.experimental import pallas as pl
from jax.experimental.pallas import tpu as pltpu
from jax.experimental.pallas import tpu_sc as plsc

PAD = 0
B, S = 4096, 50
D = 64
H2 = 256
N = B * S
EPS = 1e-12

IDX_ROWS = 512


def _tri(dtype=jnp.float32):
    r = lax.broadcasted_iota(jnp.int32, (S, S), 0)
    c = lax.broadcasted_iota(jnp.int32, (S, S), 1)
    return (r <= c).astype(dtype)


def _idx_body(src_ref, len_ref, pid2_ref):
    src = src_ref[...]
    m = src != PAD
    mf = m.astype(jnp.float32)
    pid = jnp.dot(mf, _tri(), preferred_element_type=jnp.float32)
    pid = pid.astype(jnp.int32) * m.astype(jnp.int32)
    pid2_ref[...] = jnp.maximum(len_ref[...] + 2 - pid, 0) * m.astype(jnp.int32)


def _make_ids(src, length):
    return pl.pallas_call(
        _idx_body,
        grid=(B // IDX_ROWS,),
        in_specs=[
            pl.BlockSpec((IDX_ROWS, S), lambda i: (i, 0)),
            pl.BlockSpec((IDX_ROWS, 1), lambda i: (i, 0)),
        ],
        out_specs=pl.BlockSpec((IDX_ROWS, S), lambda i: (i, 0)),
        out_shape=jax.ShapeDtypeStruct((B, S), jnp.int32),
    )(src, length.reshape(B, 1))


NW = 32
TPW = N // NW
CHUNK = 128
NCHUNK = TPW // CHUNK


def _sc_gather_body(pid2_idx, pos2_t, out, idx_v, buf, buf_e, sem):
    wid = lax.axis_index("s") * 2 + lax.axis_index("c")
    base = wid * TPW

    def chunk_body(it, carry):
        tb = base + it * CHUNK
        pltpu.sync_copy(pid2_idx.at[pl.ds(tb, CHUNK)], idx_v)
        pltpu.async_copy(pos2_t.at[idx_v], buf, sem).wait()

        def compact_body(j, carry2):
            for rr in range(4):
                for cc in range(D // 16):
                    r = 4 * j + rr
                    buf_e[r, pl.ds(cc * 16, 16)] = buf[r, pl.ds(cc * 16, 16)]
            return carry2

        lax.fori_loop(0, CHUNK // 4, compact_body, 0, unroll=False)
        pltpu.sync_copy(buf_e, out.at[pl.ds(tb, CHUNK)])
        return carry

    lax.fori_loop(0, NCHUNK, chunk_body, 0, unroll=False)


_sc_gather = functools.partial(
    pl.kernel,
    out_type=jax.ShapeDtypeStruct((N, D), jnp.float32),
    mesh=plsc.VectorSubcoreMesh(core_axis_name="c", subcore_axis_name="s"),
    scratch_types=[
        pltpu.VMEM((CHUNK,), jnp.int32),
        pltpu.VMEM((CHUNK, 2 * D), jnp.float32),
        pltpu.VMEM((CHUNK, D), jnp.float32),
        pltpu.SemaphoreType.DMA,
    ],
)(_sc_gather_body)


RB = 64
TOK = RB * S


def _main_body(src_ref, len_ref, e2_ref, aa_ref, pos_ref, g_ref, b_ref,
               w_ref, ub_ref, o_ref):
    src = src_ref[...]
    m = src != PAD
    mf = m.astype(jnp.float32)
    pid = jnp.dot(mf, _tri(), preferred_element_type=jnp.float32)
    pid_i = pid.astype(jnp.int32) * m.astype(jnp.int32)

    oh_pos = (pid_i[..., None] == lax.broadcasted_iota(
        jnp.int32, (RB, S, D), 2)).astype(jnp.float32).reshape(TOK, D)
    oh_aa = (src[..., None] == lax.broadcasted_iota(
        jnp.int32, (RB, S, 32), 2)).astype(jnp.float32).reshape(TOK, 32)
    e = (jnp.dot(oh_pos, pos_ref[...], preferred_element_type=jnp.float32)
         + jnp.dot(oh_aa, aa_ref[...], preferred_element_type=jnp.float32)
         + e2_ref[...])

    mu = jnp.mean(e, axis=1, keepdims=True)
    cent = e - mu
    var = jnp.mean(cent * cent, axis=1, keepdims=True)
    normed = cent * lax.rsqrt(var + EPS) * g_ref[...] + b_ref[...]
    o_ref[...] = (
        jnp.dot(normed, w_ref[...], preferred_element_type=jnp.float32)
        + ub_ref[...]
    )


def _main(src, length, e2, aa_pad, pos_head, ln_gamma, ln_beta, up_w, up_b):
    return pl.pallas_call(
        _main_body,
        grid=(B // RB,),
        in_specs=[
            pl.BlockSpec((RB, S), lambda i: (i, 0)),
            pl.BlockSpec((RB, 1), lambda i: (i, 0)),
            pl.BlockSpec((TOK, D), lambda i: (i, 0)),
            pl.BlockSpec((32, D), lambda i: (0, 0)),
            pl.BlockSpec((D, D), lambda i: (0, 0)),
            pl.BlockSpec((1, D), lambda i: (0, 0)),
            pl.BlockSpec((1, D), lambda i: (0, 0)),
            pl.BlockSpec((D, H2), lambda i: (0, 0)),
            pl.BlockSpec((1, H2), lambda i: (0, 0)),
        ],
        out_specs=pl.BlockSpec((TOK, H2), lambda i: (i, 0)),
        out_shape=jax.ShapeDtypeStruct((N, H2), jnp.float32),
    )(src, length.reshape(B, 1), e2, aa_pad, pos_head,
      ln_gamma.reshape(1, D), ln_beta.reshape(1, D), up_w,
      up_b.reshape(1, H2))


def kernel(src, length, aa_table, pos_table, pos2_table, ln_gamma, ln_beta,
           up_w, up_b):
    src = src.astype(jnp.int32)
    length = length.astype(jnp.int32)
    pid2 = _make_ids(src, length)
    pos2_pad = jnp.pad(pos2_table, ((0, 0), (0, D)))
    e2 = _sc_gather(pid2.reshape(N), pos2_pad)
    aa_pad = jnp.pad(aa_table, ((0, 1), (0, 0)))
    out = _main(src, length, e2, aa_pad, pos_table[:D], ln_gamma, ln_beta,
                up_w, up_b)
    return out.reshape(B, S, H2)
